# speedup vs baseline: 1.3194x; 1.3194x over previous
import functools

import jax
import jax.numpy as jnp
from jax.experimental import pallas as pl
from jax.experimental.pallas import tpu as pltpu

NUM_CLASSES = 1000
ROWS_PER_STEP = 8


def _scatter_mean_kernel(label_ref, x_ref, out_ref, counts_ref, *, rows_per_step, num_classes):
    i = pl.program_id(0)
    nsteps = pl.num_programs(0)

    @pl.when(i == 0)
    def _init():
        out_ref[...] = jnp.zeros_like(out_ref)

        def zero_counts(c, _):
            counts_ref[c] = 0
            return 0

        jax.lax.fori_loop(0, num_classes, zero_counts, 0)

    for r in range(rows_per_step):
        lbl = label_ref[i * rows_per_step + r]
        out_ref[lbl] += x_ref[r]
        counts_ref[lbl] += 1

    @pl.when(i == nsteps - 1)
    def _divide():
        def div_class(c, _):
            cnt = counts_ref[c]
            inv = 1.0 / jnp.maximum(cnt, 1).astype(jnp.float32)
            out_ref[c] = out_ref[c] * inv
            return 0

        jax.lax.fori_loop(0, num_classes, div_class, 0)


@functools.partial(jax.jit, static_argnames=("interpret",))
def _scatter_mean(x2d, label, *, interpret=False):
    n, f = x2d.shape
    sub = f // 128
    x3 = x2d.reshape(n, sub, 128)
    grid = n // ROWS_PER_STEP

    out = pl.pallas_call(
        functools.partial(
            _scatter_mean_kernel,
            rows_per_step=ROWS_PER_STEP,
            num_classes=NUM_CLASSES,
        ),
        grid_spec=pltpu.PrefetchScalarGridSpec(
            num_scalar_prefetch=1,
            grid=(grid,),
            in_specs=[
                pl.BlockSpec((ROWS_PER_STEP, sub, 128), lambda i, lbl: (i, 0, 0)),
            ],
            out_specs=pl.BlockSpec((NUM_CLASSES, sub, 128), lambda i, lbl: (0, 0, 0)),
            scratch_shapes=[pltpu.SMEM((NUM_CLASSES,), jnp.int32)],
        ),
        out_shape=jax.ShapeDtypeStruct((NUM_CLASSES, sub, 128), jnp.float32),
        interpret=interpret,
    )(label.astype(jnp.int32), x3)
    return out


def kernel(x, label):
    n, c, h, w = x.shape
    x2d = x.reshape(n, c * h * w)
    out = _scatter_mean(x2d, label)
    return out.reshape(NUM_CLASSES, c, h, w)


# speedup vs baseline: 1.6500x; 1.2506x over previous
import functools

import jax
import jax.numpy as jnp
from jax.experimental import pallas as pl
from jax.experimental.pallas import tpu as pltpu

NUM_CLASSES = 1000
ROWS_PER_STEP = 64


def _scatter_mean_kernel(label_ref, x_ref, out_ref, counts_ref, *, rows_per_step, num_classes):
    i = pl.program_id(0)
    nsteps = pl.num_programs(0)

    @pl.when(i == 0)
    def _init():
        out_ref[...] = jnp.zeros_like(out_ref)

        def zero_counts(c, _):
            counts_ref[c] = 0
            return 0

        jax.lax.fori_loop(0, num_classes, zero_counts, 0)

    for r in range(rows_per_step):
        lbl = label_ref[i * rows_per_step + r]
        out_ref[lbl] += x_ref[r]
        counts_ref[lbl] += 1

    @pl.when(i == nsteps - 1)
    def _divide():
        def div_class(c, _):
            cnt = counts_ref[c]
            inv = 1.0 / jnp.maximum(cnt, 1).astype(jnp.float32)
            out_ref[c] = out_ref[c] * inv
            return 0

        jax.lax.fori_loop(0, num_classes, div_class, 0)


@functools.partial(jax.jit, static_argnames=("interpret",))
def _scatter_mean(x2d, label, *, interpret=False):
    n, f = x2d.shape
    sub = f // 128
    x3 = x2d.reshape(n, sub, 128)
    grid = n // ROWS_PER_STEP

    out = pl.pallas_call(
        functools.partial(
            _scatter_mean_kernel,
            rows_per_step=ROWS_PER_STEP,
            num_classes=NUM_CLASSES,
        ),
        grid_spec=pltpu.PrefetchScalarGridSpec(
            num_scalar_prefetch=1,
            grid=(grid,),
            in_specs=[
                pl.BlockSpec((ROWS_PER_STEP, sub, 128), lambda i, lbl: (i, 0, 0)),
            ],
            out_specs=pl.BlockSpec((NUM_CLASSES, sub, 128), lambda i, lbl: (0, 0, 0)),
            scratch_shapes=[pltpu.SMEM((NUM_CLASSES,), jnp.int32)],
        ),
        out_shape=jax.ShapeDtypeStruct((NUM_CLASSES, sub, 128), jnp.float32),
        interpret=interpret,
    )(label.astype(jnp.int32), x3)
    return out


def kernel(x, label):
    n, c, h, w = x.shape
    x2d = x.reshape(n, c * h * w)
    out = _scatter_mean(x2d, label)
    return out.reshape(NUM_CLASSES, c, h, w)


# speedup vs baseline: 1.6515x; 1.0009x over previous
import functools

import jax
import jax.numpy as jnp
from jax.experimental import pallas as pl
from jax.experimental.pallas import tpu as pltpu

NUM_CLASSES = 1000
ROWS_PER_STEP = 64


def _scatter_mean_kernel(label_ref, x_ref, out_ref, counts_ref, *, rows_per_step, num_classes):
    i = pl.program_id(0)
    nsteps = pl.num_programs(0)

    @pl.when(i == 0)
    def _init():
        out_ref[...] = jnp.zeros_like(out_ref)

        def zero_counts(c, _):
            counts_ref[c] = 0
            return 0

        jax.lax.fori_loop(0, num_classes, zero_counts, 0)

    for r in range(rows_per_step):
        lbl = label_ref[i * rows_per_step + r]
        out_ref[lbl] += x_ref[r]
        counts_ref[lbl] += 1

    @pl.when(i == nsteps - 1)
    def _divide():
        def div_class(c, _):
            cnt = counts_ref[c]
            inv = 1.0 / jnp.maximum(cnt, 1).astype(jnp.float32)
            out_ref[c] = out_ref[c] * inv
            return 0

        jax.lax.fori_loop(0, num_classes, div_class, 0)


@functools.partial(jax.jit, static_argnames=("interpret",))
def _scatter_mean(x3, label, *, interpret=False):
    n, sub, _ = x3.shape
    grid = n // ROWS_PER_STEP

    out = pl.pallas_call(
        functools.partial(
            _scatter_mean_kernel,
            rows_per_step=ROWS_PER_STEP,
            num_classes=NUM_CLASSES,
        ),
        grid_spec=pltpu.PrefetchScalarGridSpec(
            num_scalar_prefetch=1,
            grid=(grid,),
            in_specs=[
                pl.BlockSpec((ROWS_PER_STEP, sub, 128), lambda i, lbl: (i, 0, 0)),
            ],
            out_specs=pl.BlockSpec((NUM_CLASSES, sub, 128), lambda i, lbl: (0, 0, 0)),
            scratch_shapes=[pltpu.SMEM((NUM_CLASSES,), jnp.int32)],
        ),
        out_shape=jax.ShapeDtypeStruct((NUM_CLASSES, sub, 128), jnp.float32),
        interpret=interpret,
    )(label.astype(jnp.int32), x3)
    return out


def kernel(x, label):
    n, c, h, w = x.shape
    f = c * h * w
    x3 = x.reshape(n, f // 128, 128)
    out = _scatter_mean(x3, label)
    return out.reshape(NUM_CLASSES, c, h, w)


# speedup vs baseline: 11.2752x; 6.8273x over previous
import functools

import jax
import jax.numpy as jnp
from jax.experimental import pallas as pl
from jax.experimental.pallas import tpu as pltpu

NUM_CLASSES = 1000
CLASS_PAD = 1024
M_BLOCK = 896


def _onehot_matmul_kernel(x_ref, lbl_ref, out_ref, p_ref, inv_ref):
    i = pl.program_id(0)

    @pl.when(i == 0)
    def _build_p():
        lbl = lbl_ref[...]
        classes = jax.lax.broadcasted_iota(jnp.int32, (1, CLASS_PAD), 1)
        onehot = lbl == classes
        p_ref[...] = onehot.astype(jnp.bfloat16)
        counts = jnp.sum(onehot.astype(jnp.float32), axis=0, keepdims=True)
        inv_ref[...] = 1.0 / jnp.maximum(counts, 1.0)

    xb = x_ref[...].astype(jnp.bfloat16)
    acc = jnp.dot(xb, p_ref[...], preferred_element_type=jnp.float32)
    out_ref[...] = (acc * inv_ref[...])[:, :NUM_CLASSES]


@jax.jit
def _scatter_mean(x2, lbl2):
    m, n = x2.shape
    grid = m // M_BLOCK

    out = pl.pallas_call(
        _onehot_matmul_kernel,
        grid=(grid,),
        in_specs=[
            pl.BlockSpec((M_BLOCK, n), lambda i: (i, 0)),
            pl.BlockSpec((n, 1), lambda i: (0, 0)),
        ],
        out_specs=pl.BlockSpec((M_BLOCK, NUM_CLASSES), lambda i: (i, 0)),
        out_shape=jax.ShapeDtypeStruct((m, NUM_CLASSES), jnp.float32),
        scratch_shapes=[
            pltpu.VMEM((n, CLASS_PAD), jnp.bfloat16),
            pltpu.VMEM((1, CLASS_PAD), jnp.float32),
        ],
        compiler_params=pltpu.CompilerParams(
            dimension_semantics=("arbitrary",),
        ),
    )(x2, lbl2)
    return out


def kernel(x, label):
    n, c, h, w = x.shape
    f = c * h * w
    x2 = x.transpose(2, 3, 1, 0).reshape(f, n)
    lbl2 = label.astype(jnp.int32).reshape(n, 1)
    out2 = _scatter_mean(x2, lbl2)
    return out2.reshape(h, w, c, NUM_CLASSES).transpose(3, 2, 0, 1)
